# baseline (speedup 1.0000x reference)
import functools

import jax
import jax.numpy as jnp
from jax import lax
from jax.experimental import pallas as pl
from jax.experimental.pallas import tpu as pltpu
from jax.experimental.pallas import tpu_sc as plsc

_EMB = 64
_NC, _NS = 2, 16
_NW = _NC * _NS
_CHUNK = 128


@functools.partial(jax.jit, static_argnums=(2,))
def _gather(idx, table, n_rows):
    nb = n_rows // _NW
    nch = nb // _CHUNK
    mesh = plsc.VectorSubcoreMesh(core_axis_name="c", subcore_axis_name="s")

    @functools.partial(
        pl.kernel,
        out_type=jax.ShapeDtypeStruct((n_rows, _EMB), jnp.float32),
        mesh=mesh,
        scratch_types=[
            pltpu.VMEM((nch, _CHUNK), jnp.int32),
            pltpu.VMEM((_CHUNK, _EMB), jnp.float32),
            pltpu.SemaphoreType.DMA,
        ],
        compiler_params=pltpu.CompilerParams(use_tc_tiling_on_sc=False),
    )
    def gather_kernel(idx_hbm, table_hbm, out_hbm, idx_v, rows_v, sem):
        wid = lax.axis_index("s") * _NC + lax.axis_index("c")
        base_ch = wid * nch
        pltpu.sync_copy(idx_hbm.at[pl.ds(base_ch, nch)], idx_v)

        def body(j, carry):
            pltpu.async_copy(table_hbm.at[idx_v.at[j]], rows_v, sem).wait()
            pltpu.sync_copy(
                rows_v, out_hbm.at[pl.ds((base_ch + j) * _CHUNK, _CHUNK)]
            )
            return carry

        lax.fori_loop(0, nch, body, 0)

    return gather_kernel(idx, table)


def kernel(indices, table):
    b, l = indices.shape
    n = b * l
    idx = indices.reshape(n // _CHUNK, _CHUNK).astype(jnp.int32)
    out = _gather(idx, table, n)
    return out.reshape(b, l, _EMB)


# speedup vs baseline: 1.1135x; 1.1135x over previous
import functools

import jax
import jax.numpy as jnp
from jax import lax
from jax.experimental import pallas as pl
from jax.experimental.pallas import tpu as pltpu
from jax.experimental.pallas import tpu_sc as plsc

_EMB = 64
_NC, _NS = 2, 16
_NW = _NC * _NS
_CHUNK = 128
_GRP = 4
_GROWS = _GRP * _CHUNK


@functools.partial(jax.jit, static_argnums=(2,))
def _gather(idx, table, n_rows):
    nb = n_rows // _NW
    nch = nb // _CHUNK
    ngrp = nch // _GRP
    mesh = plsc.VectorSubcoreMesh(core_axis_name="c", subcore_axis_name="s")

    @functools.partial(
        pl.kernel,
        out_type=jax.ShapeDtypeStruct((n_rows, _EMB), jnp.float32),
        mesh=mesh,
        scratch_types=[
            pltpu.VMEM((nch, _CHUNK), jnp.int32),
            pltpu.VMEM((_GROWS, _EMB), jnp.float32),
            pltpu.VMEM((_GROWS, _EMB), jnp.float32),
            pltpu.SemaphoreType.DMA,
            pltpu.SemaphoreType.DMA,
            pltpu.SemaphoreType.DMA,
            pltpu.SemaphoreType.DMA,
        ],
        compiler_params=pltpu.CompilerParams(use_tc_tiling_on_sc=False),
    )
    def gather_kernel(idx_hbm, table_hbm, out_hbm, idx_v, buf0, buf1,
                      sg0, sg1, sw0, sw1):
        wid = lax.axis_index("s") * _NC + lax.axis_index("c")
        base_ch = wid * nch
        base_row = base_ch * _CHUNK
        pltpu.sync_copy(idx_hbm.at[pl.ds(base_ch, nch)], idx_v)

        slots = ((buf0, sg0, sw0), (buf1, sg1, sw1))

        def fire_group(g, buf, sg):
            for k in range(_GRP):
                pltpu.async_copy(
                    table_hbm.at[idx_v.at[g * _GRP + k]],
                    buf.at[pl.ds(k * _CHUNK, _CHUNK)],
                    sg,
                )

        def drain_group(buf, sg):
            pltpu.make_async_copy(
                table_hbm.at[pl.ds(0, _GROWS)], buf, sg
            ).wait()

        def start_write(g, buf, sw):
            pltpu.async_copy(
                buf, out_hbm.at[pl.ds(base_row + g * _GROWS, _GROWS)], sw
            )

        def wait_write(buf, sw):
            pltpu.make_async_copy(
                buf, out_hbm.at[pl.ds(base_row, _GROWS)], sw
            ).wait()

        fire_group(0, buf0, sg0)

        @pl.loop(0, ngrp // 2)
        def _(g2):
            for p in range(2):
                g = g2 * 2 + p
                buf, sg, sw = slots[p]
                obuf, osg, osw = slots[1 - p]
                drain_group(buf, sg)
                @pl.when(g >= 1)
                def _():
                    wait_write(obuf, osw)

                @pl.when(g + 1 < ngrp)
                def _():
                    fire_group(g + 1, obuf, osg)

                start_write(g, buf, sw)

        lbuf, _lsg, lsw = slots[(ngrp - 1) % 2]
        wait_write(lbuf, lsw)

    return gather_kernel(idx, table)


def kernel(indices, table):
    b, l = indices.shape
    n = b * l
    idx = indices.reshape(n // _CHUNK, _CHUNK).astype(jnp.int32)
    out = _gather(idx, table, n)
    return out.reshape(b, l, _EMB)
